# baseline (speedup 1.0000x reference)
import jax
import jax.numpy as jnp
from jax.experimental import pallas as pl

_BLOCK_ROWS = 256


def _body(t_ref, o_ref):
    o_ref[...] = t_ref[...][None]


def kernel(x, table):
    bs, seq_l, d = x.shape
    return pl.pallas_call(
        _body,
        grid=(seq_l // _BLOCK_ROWS, bs),
        in_specs=[pl.BlockSpec((_BLOCK_ROWS, d), lambda i, b: (i, 0))],
        out_specs=pl.BlockSpec((1, _BLOCK_ROWS, d), lambda i, b: (b, i, 0)),
        out_shape=jax.ShapeDtypeStruct((bs, seq_l, d), jnp.float32),
    )(table[:seq_l])


# speedup vs baseline: 1.5339x; 1.5339x over previous
import jax
import jax.numpy as jnp
from jax.experimental import pallas as pl

_BLOCK_ROWS = 256


def _body(x_ref, o_ref):
    o_ref[...] = x_ref[...]


def kernel(x, table):
    bs, seq_l, d = x.shape
    return pl.pallas_call(
        _body,
        grid=(seq_l // _BLOCK_ROWS,),
        in_specs=[pl.BlockSpec((bs, _BLOCK_ROWS, d), lambda i: (0, i, 0))],
        out_specs=pl.BlockSpec((bs, _BLOCK_ROWS, d), lambda i: (0, i, 0)),
        out_shape=jax.ShapeDtypeStruct((bs, seq_l, d), jnp.float32),
    )(x)


# speedup vs baseline: 2.2785x; 1.4854x over previous
import jax
import jax.numpy as jnp
from jax.experimental import pallas as pl

_BLOCK_ROWS = 256


def _body(t_ref, o_ref):
    o_ref[...] = jnp.broadcast_to(t_ref[...][None], o_ref.shape)


def kernel(x, table):
    bs, seq_l, d = x.shape
    return pl.pallas_call(
        _body,
        grid=(seq_l // _BLOCK_ROWS,),
        in_specs=[pl.BlockSpec((_BLOCK_ROWS, d), lambda i: (i, 0))],
        out_specs=pl.BlockSpec((bs, _BLOCK_ROWS, d), lambda i: (0, i, 0)),
        out_shape=jax.ShapeDtypeStruct((bs, seq_l, d), jnp.float32),
    )(table)


# speedup vs baseline: 2.6400x; 1.1586x over previous
import jax
import jax.numpy as jnp
from jax.experimental import pallas as pl
from jax.experimental.pallas import tpu as pltpu

_CHUNK_ROWS = 256


def _make_kernel(bs, seq_l, d):
    n_chunks = seq_l // _CHUNK_ROWS

    def body(t_hbm, o_hbm, buf, rsems, wsem):
        reads = []
        for c in range(n_chunks):
            sl = pl.ds(c * _CHUNK_ROWS, _CHUNK_ROWS)
            cp = pltpu.make_async_copy(t_hbm.at[sl], buf.at[sl], rsems.at[c])
            cp.start()
            reads.append(cp)
        writes = []
        for c in range(n_chunks):
            reads[c].wait()
            sl = pl.ds(c * _CHUNK_ROWS, _CHUNK_ROWS)
            for b in range(bs):
                w = pltpu.make_async_copy(buf.at[sl], o_hbm.at[b, sl], wsem)
                w.start()
                writes.append(w)
        for w in writes:
            w.wait()

    return pl.pallas_call(
        body,
        in_specs=[pl.BlockSpec(memory_space=pltpu.MemorySpace.HBM)],
        out_specs=pl.BlockSpec(memory_space=pltpu.MemorySpace.HBM),
        out_shape=jax.ShapeDtypeStruct((bs, seq_l, d), jnp.float32),
        scratch_shapes=[
            pltpu.VMEM((seq_l, d), jnp.float32),
            pltpu.SemaphoreType.DMA((n_chunks,)),
            pltpu.SemaphoreType.DMA,
        ],
    )


def kernel(x, table):
    bs, seq_l, d = x.shape
    return _make_kernel(bs, seq_l, d)(table)


# speedup vs baseline: 2.6545x; 1.0055x over previous
import jax
import jax.numpy as jnp
from jax.experimental import pallas as pl
from jax.experimental.pallas import tpu as pltpu

_CHUNK_ROWS = 512


def _make_kernel(bs, seq_l, d):
    n_chunks = seq_l // _CHUNK_ROWS

    def body(t_hbm, o_hbm, buf, rsems, wsem):
        reads = []
        for c in range(n_chunks):
            sl = pl.ds(c * _CHUNK_ROWS, _CHUNK_ROWS)
            cp = pltpu.make_async_copy(t_hbm.at[sl], buf.at[sl], rsems.at[c])
            cp.start()
            reads.append(cp)
        writes = []
        for c in range(n_chunks):
            reads[c].wait()
            sl = pl.ds(c * _CHUNK_ROWS, _CHUNK_ROWS)
            for b in range(bs):
                w = pltpu.make_async_copy(buf.at[sl], o_hbm.at[b, sl], wsem)
                w.start()
                writes.append(w)
        for w in writes:
            w.wait()

    return pl.pallas_call(
        body,
        in_specs=[pl.BlockSpec(memory_space=pltpu.MemorySpace.HBM)],
        out_specs=pl.BlockSpec(memory_space=pltpu.MemorySpace.HBM),
        out_shape=jax.ShapeDtypeStruct((bs, seq_l, d), jnp.float32),
        scratch_shapes=[
            pltpu.VMEM((seq_l, d), jnp.float32),
            pltpu.SemaphoreType.DMA((n_chunks,)),
            pltpu.SemaphoreType.DMA,
        ],
    )


def kernel(x, table):
    bs, seq_l, d = x.shape
    return _make_kernel(bs, seq_l, d)(table)


# speedup vs baseline: 2.6756x; 1.0079x over previous
import jax
import jax.numpy as jnp
from jax.experimental import pallas as pl
from jax.experimental.pallas import tpu as pltpu

_CHUNK_ROWS = 1024


def _make_kernel(bs, seq_l, d):
    n_chunks = seq_l // _CHUNK_ROWS

    def body(t_hbm, o_hbm, buf, rsems, wsem):
        reads = []
        for c in range(n_chunks):
            sl = pl.ds(c * _CHUNK_ROWS, _CHUNK_ROWS)
            cp = pltpu.make_async_copy(t_hbm.at[sl], buf.at[sl], rsems.at[c])
            cp.start()
            reads.append(cp)
        writes = []
        for c in range(n_chunks):
            reads[c].wait()
            sl = pl.ds(c * _CHUNK_ROWS, _CHUNK_ROWS)
            for b in range(bs):
                w = pltpu.make_async_copy(buf.at[sl], o_hbm.at[b, sl], wsem)
                w.start()
                writes.append(w)
        for w in writes:
            w.wait()

    return pl.pallas_call(
        body,
        in_specs=[pl.BlockSpec(memory_space=pltpu.MemorySpace.HBM)],
        out_specs=pl.BlockSpec(memory_space=pltpu.MemorySpace.HBM),
        out_shape=jax.ShapeDtypeStruct((bs, seq_l, d), jnp.float32),
        scratch_shapes=[
            pltpu.VMEM((seq_l, d), jnp.float32),
            pltpu.SemaphoreType.DMA((n_chunks,)),
            pltpu.SemaphoreType.DMA,
        ],
    )


def kernel(x, table):
    bs, seq_l, d = x.shape
    return _make_kernel(bs, seq_l, d)(table)
